# baseline (speedup 1.0000x reference)
import jax
import jax.numpy as jnp
from jax.experimental import pallas as pl
from jax.experimental.pallas import tpu as pltpu

_P, _E = 1024, 768
_CH = 8192
_NBUF = 2


def _pipe_kernel(x_hbm, pos_hbm, o_hbm, bufs, pos_v, in_sems, out_sems, pos_sem):
    n_rows = x_hbm.shape[0]
    K = n_rows // _CH

    def in_copy(k, j):
        return pltpu.make_async_copy(
            x_hbm.at[pl.ds(k * _CH, _CH), :], bufs.at[j], in_sems.at[j])

    def out_copy(k, j):
        return pltpu.make_async_copy(
            bufs.at[j], o_hbm.at[pl.ds(k * _CH, _CH), :], out_sems.at[j])

    pltpu.make_async_copy(pos_hbm, pos_v, pos_sem).start()
    for j in range(_NBUF - 1):
        in_copy(j, j).start()
    pltpu.make_async_copy(pos_hbm, pos_v, pos_sem).wait()

    def step(k, carry):
        j = jax.lax.rem(k, _NBUF)
        in_copy(k, j).wait()
        buf = bufs.at[j]
        for t in range(_CH // _P):
            sl = pl.ds(t * _P, _P)
            buf[sl, :] = buf[sl, :] + pos_v[...]
        out_copy(k, j).start()
        kn = k + _NBUF - 1
        jn = jax.lax.rem(kn, _NBUF)

        @pl.when(jnp.logical_and(k >= 1, kn < K))
        def _():
            out_copy(k - 1, jn).wait()

        @pl.when(kn < K)
        def _():
            in_copy(kn, jn).start()

        return carry

    jax.lax.fori_loop(0, K, step, 0)
    for d in range(_NBUF):
        kd = K - _NBUF + d
        out_copy(kd, kd % _NBUF).wait()


def kernel(x, pos_table):
    B, P, E = x.shape
    x2 = x.reshape(B * P, E)
    out = pl.pallas_call(
        _pipe_kernel,
        in_specs=[
            pl.BlockSpec(memory_space=pltpu.HBM),
            pl.BlockSpec(memory_space=pltpu.HBM),
        ],
        out_specs=pl.BlockSpec(memory_space=pltpu.HBM),
        out_shape=jax.ShapeDtypeStruct((B * P, E), x.dtype),
        scratch_shapes=[
            pltpu.VMEM((_NBUF, _CH, _E), jnp.float32),
            pltpu.VMEM((_P, _E), jnp.float32),
            pltpu.SemaphoreType.DMA((_NBUF,)),
            pltpu.SemaphoreType.DMA((_NBUF,)),
            pltpu.SemaphoreType.DMA,
        ],
    )(x2, pos_table)
    return out.reshape(B, P, E)


# speedup vs baseline: 1.2543x; 1.2543x over previous
import jax
import jax.numpy as jnp
from jax.experimental import pallas as pl
from jax.experimental.pallas import tpu as pltpu

_P, _E = 1024, 768
_CH = 4096
_NBUF = 5


def _pipe_kernel(x_hbm, pos_hbm, o_hbm, bufs, pos_v, in_sems, out_sems, pos_sem):
    n_rows = x_hbm.shape[0]
    K = n_rows // _CH

    def in_copy(k, j):
        return pltpu.make_async_copy(
            x_hbm.at[pl.ds(k * _CH, _CH), :], bufs.at[j], in_sems.at[j])

    def out_copy(k, j):
        return pltpu.make_async_copy(
            bufs.at[j], o_hbm.at[pl.ds(k * _CH, _CH), :], out_sems.at[j])

    pltpu.make_async_copy(pos_hbm, pos_v, pos_sem).start()
    for j in range(_NBUF - 1):
        in_copy(j, j).start()
    pltpu.make_async_copy(pos_hbm, pos_v, pos_sem).wait()

    def step(k, carry):
        j = jax.lax.rem(k, _NBUF)
        in_copy(k, j).wait()
        buf = bufs.at[j]
        for t in range(_CH // _P):
            sl = pl.ds(t * _P, _P)
            buf[sl, :] = buf[sl, :] + pos_v[...]
        out_copy(k, j).start()
        kn = k + _NBUF - 1
        jn = jax.lax.rem(kn, _NBUF)

        @pl.when(jnp.logical_and(k >= 1, kn < K))
        def _():
            out_copy(k - 1, jn).wait()

        @pl.when(kn < K)
        def _():
            in_copy(kn, jn).start()

        return carry

    jax.lax.fori_loop(0, K, step, 0)
    for d in range(_NBUF):
        kd = K - _NBUF + d
        out_copy(kd, kd % _NBUF).wait()


def kernel(x, pos_table):
    B, P, E = x.shape
    x2 = x.reshape(B * P, E)
    out = pl.pallas_call(
        _pipe_kernel,
        in_specs=[
            pl.BlockSpec(memory_space=pltpu.HBM),
            pl.BlockSpec(memory_space=pltpu.HBM),
        ],
        out_specs=pl.BlockSpec(memory_space=pltpu.HBM),
        out_shape=jax.ShapeDtypeStruct((B * P, E), x.dtype),
        scratch_shapes=[
            pltpu.VMEM((_NBUF, _CH, _E), jnp.float32),
            pltpu.VMEM((_P, _E), jnp.float32),
            pltpu.SemaphoreType.DMA((_NBUF,)),
            pltpu.SemaphoreType.DMA((_NBUF,)),
            pltpu.SemaphoreType.DMA,
        ],
        compiler_params=pltpu.CompilerParams(
            vmem_limit_bytes=66_584_576,
        ),
    )(x2, pos_table)
    return out.reshape(B, P, E)
